# Initial kernel scaffold; baseline (speedup 1.0000x reference)
import jax
import jax.numpy as jnp
from jax.experimental import pallas as pl


def kernel(prop, dist, ind_1, ind_2, params):
    raise NotImplementedError("write your pallas kernel here")


# speedup vs baseline: 1.6785x; 1.6785x over previous
import functools

import jax
import jax.numpy as jnp
import numpy as np
from jax import lax
from jax.experimental import pallas as pl
from jax.experimental.pallas import tpu as pltpu
from jax.experimental.pallas import tpu_sc as plsc

N = 50000
E = 800000
D = 16
NB = 4
RC = 4.0
DEPTH = 4

NC = 2
NS = 16
NW = NC * NS

E_PAD = 802816
G2 = 2 * E_PAD
N_PAD = 51200
DUMP = N

G_PER_TILE = G2 // NW
S_PER_TILE = E_PAD // NW
NG_STREAMS = 392
NS_STREAMS = 196
GK = 8
NGRP = NG_STREAMS // GK
ROWS_PT = N_PAD // NS
STAGE = 320

BE = 8192
BN = 2048

def _gather_body(p_hbm, idx_hbm, out_hbm, p_sh, stage_v, idx_v, rows_v, sem):
    c = lax.axis_index("c")
    s = lax.axis_index("s")
    wid = c * NS + s

    def load_chunk(i, carry):
        r0 = s * ROWS_PT + i * STAGE
        pltpu.sync_copy(p_hbm.at[pl.ds(r0, STAGE)], stage_v)
        pltpu.sync_copy(stage_v, p_sh.at[pl.ds(r0, STAGE)])
        return carry

    lax.fori_loop(0, ROWS_PT // STAGE, load_chunk, 0)
    plsc.subcore_barrier()

    pltpu.sync_copy(idx_hbm.at[wid], idx_v)

    def grp(g, carry):
        cps = []
        for k in range(GK):
            cp = pltpu.async_copy(
                p_sh.at[idx_v.at[g * GK + k]],
                rows_v.at[pl.ds(k * 128, 128)],
                sem,
            )
            cps.append(cp)
        for cp in cps:
            cp.wait()
        pltpu.sync_copy(
            rows_v, out_hbm.at[pl.ds(wid * G_PER_TILE + g * (GK * 128), GK * 128)]
        )
        return carry

    lax.fori_loop(0, NGRP, grp, 0)


@functools.lru_cache(maxsize=None)
def _sc_kernels():
    mesh = plsc.VectorSubcoreMesh(
        core_axis_name="c", subcore_axis_name="s", num_cores=NC, num_subcores=NS
    )
    sc_params = pltpu.CompilerParams(use_tc_tiling_on_sc=False)
    gather = pl.kernel(
        _gather_body,
        out_type=jax.ShapeDtypeStruct((G2, D), jnp.float32),
        mesh=mesh,
        compiler_params=sc_params,
        scratch_types=[
            pltpu.VMEM_SHARED((N_PAD, D), jnp.float32),
            pltpu.VMEM((STAGE, D), jnp.float32),
            pltpu.VMEM((NG_STREAMS, 128), jnp.int32),
            pltpu.VMEM((GK * 128, D), jnp.float32),
            pltpu.SemaphoreType.DMA,
        ],
    )
    scatter = pl.kernel(
        _scatter_body,
        out_type=jax.ShapeDtypeStruct((NC, N_PAD, D), jnp.float32),
        mesh=mesh,
        compiler_params=sc_params,
        scratch_types=[
            pltpu.VMEM_SHARED((N_PAD, D), jnp.float32),
            pltpu.VMEM((STAGE, D), jnp.float32),
            pltpu.VMEM((NS_STREAMS, 128), jnp.int32),
            pltpu.VMEM((128, D), jnp.float32),
        ],
    )
    return gather, scatter


def _gather_call(p, gidx):
    return _sc_kernels()[0](p, gidx)


def _scatter_body(inter_hbm, idx_hbm, zeros_hbm, out_hbm, acc_sh, stage_v, idx_v, rows_v):
    c = lax.axis_index("c")
    s = lax.axis_index("s")
    wid = c * NS + s

    def zero_chunk(i, carry):
        r0 = s * ROWS_PT + i * STAGE
        pltpu.sync_copy(zeros_hbm.at[pl.ds(r0, STAGE)], stage_v)
        pltpu.sync_copy(stage_v, acc_sh.at[pl.ds(r0, STAGE)])
        return carry

    lax.fori_loop(0, ROWS_PT // STAGE, zero_chunk, 0)
    plsc.subcore_barrier()

    pltpu.sync_copy(idx_hbm.at[wid], idx_v)

    def grp(g, carry):
        pltpu.sync_copy(inter_hbm.at[pl.ds(wid * S_PER_TILE + g * 128, 128)], rows_v)
        pltpu.sync_copy(rows_v, acc_sh.at[idx_v.at[g]], add=True)
        return carry

    lax.fori_loop(0, NS_STREAMS, grp, 0)
    plsc.subcore_barrier()

    def dump_chunk(i, carry):
        r0 = s * ROWS_PT + i * STAGE
        pltpu.sync_copy(acc_sh.at[pl.ds(r0, STAGE)], stage_v)
        pltpu.sync_copy(stage_v, out_hbm.at[c].at[pl.ds(r0, STAGE)])
        return carry

    lax.fori_loop(0, ROWS_PT // STAGE, dump_chunk, 0)


def _scatter_call(inter, sidx, zeros_np):
    return _sc_kernels()[1](inter, sidx, zeros_np)


def _edge_body(pg_ref, dist_ref, w1, b1, w2, b2, v1, v2, o_ref):
    x = pg_ref[...]
    h = jnp.tanh(x @ w1[...] + b1[...])
    h = jnp.tanh(h @ w2[...] + b2[...])
    fc = 0.5 * (jnp.cos((np.pi / RC) * dist_ref[...]) + 1.0)
    f2 = fc * fc
    f3 = f2 * fc
    f4 = f3 * fc
    it = h[:, 0:16] * fc + h[:, 16:32] * f2 + h[:, 32:48] * f3 + h[:, 48:64] * f4
    it = jnp.tanh(it @ v1[...])
    it = jnp.tanh(it @ v2[...])
    o_ref[...] = it


def _edge_call(pg, dist2d, w1, b1, w2p, b2p, v1, v2):
    full = pl.BlockSpec(memory_space=pltpu.MemorySpace.VMEM)
    return pl.pallas_call(
        _edge_body,
        grid=(E_PAD // BE,),
        in_specs=[
            pl.BlockSpec((BE, 2 * D), lambda i: (i, 0)),
            pl.BlockSpec((BE, 1), lambda i: (i, 0)),
            full, full, full, full, full, full,
        ],
        out_specs=pl.BlockSpec((BE, D), lambda i: (i, 0)),
        out_shape=jax.ShapeDtypeStruct((E_PAD, D), jnp.float32),
        compiler_params=pltpu.CompilerParams(
            dimension_semantics=("arbitrary",),
        ),
    )(pg, dist2d, w1, b1, w2p, b2p, v1, v2)


def _node_body(parts_ref, prop_ref, oacc_ref, u1, c1, u2, c2, wout,
               pw1, pb1, pw2, pb2, prop_o, pnext_o, out_o):
    npv = parts_ref[0] + parts_ref[1]
    h = npv @ u1[...] + c1[...]
    h = h @ u2[...] + c2[...]
    out_o[...] = oacc_ref[...] + h @ wout[...]
    pr = prop_ref[...] + npv
    prop_o[...] = pr
    q = jnp.tanh(pr @ pw1[...] + pb1[...])
    pnext_o[...] = jnp.tanh(q @ pw2[...] + pb2[...])


def _node_last_body(parts_ref, oacc_ref, u1, c1, u2, c2, wout, out_o):
    npv = parts_ref[0] + parts_ref[1]
    h = npv @ u1[...] + c1[...]
    h = h @ u2[...] + c2[...]
    out_o[...] = oacc_ref[...] + h @ wout[...]


def _node_call(parts, prop, oacc, u1, c1, u2, c2, wout, pw1, pb1, pw2, pb2):
    full = pl.BlockSpec(memory_space=pltpu.MemorySpace.VMEM)
    return pl.pallas_call(
        _node_body,
        grid=(N_PAD // BN,),
        in_specs=[
            pl.BlockSpec((NC, BN, D), lambda i: (0, i, 0)),
            pl.BlockSpec((BN, D), lambda i: (i, 0)),
            pl.BlockSpec((BN, 1), lambda i: (i, 0)),
            full, full, full, full, full, full, full, full, full,
        ],
        out_specs=[
            pl.BlockSpec((BN, D), lambda i: (i, 0)),
            pl.BlockSpec((BN, D), lambda i: (i, 0)),
            pl.BlockSpec((BN, 1), lambda i: (i, 0)),
        ],
        out_shape=[
            jax.ShapeDtypeStruct((N_PAD, D), jnp.float32),
            jax.ShapeDtypeStruct((N_PAD, D), jnp.float32),
            jax.ShapeDtypeStruct((N_PAD, 1), jnp.float32),
        ],
        compiler_params=pltpu.CompilerParams(
            dimension_semantics=("arbitrary",),
        ),
    )(parts, prop, oacc, u1, c1, u2, c2, wout, pw1, pb1, pw2, pb2)


def _node_last_call(parts, oacc, u1, c1, u2, c2, wout):
    full = pl.BlockSpec(memory_space=pltpu.MemorySpace.VMEM)
    return pl.pallas_call(
        _node_last_body,
        grid=(N_PAD // BN,),
        in_specs=[
            pl.BlockSpec((NC, BN, D), lambda i: (0, i, 0)),
            pl.BlockSpec((BN, 1), lambda i: (i, 0)),
            full, full, full, full, full,
        ],
        out_specs=pl.BlockSpec((BN, 1), lambda i: (i, 0)),
        out_shape=jax.ShapeDtypeStruct((N_PAD, 1), jnp.float32),
        compiler_params=pltpu.CompilerParams(
            dimension_semantics=("arbitrary",),
        ),
    )(parts, oacc, u1, c1, u2, c2, wout)


def _edge_weights(blk):
    (w1, b1), (w2, b2) = blk["pi"]
    w2p = w2.reshape(D, D, NB).transpose(0, 2, 1).reshape(D, D * NB)
    b2p = b2.reshape(D, NB).T.reshape(D * NB)
    (v1, _), (v2, _) = blk["ii"]
    return (w1, b1.reshape(1, D), w2p, b2p.reshape(1, D * NB), v1, v2)


def _out_weights(o):
    (u1, c1), (u2, c2) = o["ff"]
    return (u1, c1.reshape(1, D), u2, c2.reshape(1, D), o["Wout"])


def kernel(prop, dist, ind_1, ind_2, params):
    del ind_1
    f32 = jnp.float32
    prop_pad = jnp.pad(prop.astype(f32), ((0, N_PAD - N), (0, 0)))
    dist2d = jnp.pad(dist.astype(f32), (0, E_PAD - E)).reshape(E_PAD, 1)
    gidx = jnp.pad(
        ind_2.reshape(-1).astype(jnp.int32), (0, G2 - 2 * E)
    ).reshape(NW, NG_STREAMS, 128)
    sidx = jnp.concatenate(
        [ind_2[:, 0].astype(jnp.int32), jnp.full((E_PAD - E,), DUMP, jnp.int32)]
    ).reshape(NW, NS_STREAMS, 128)
    zeros_np = jnp.zeros((N_PAD, D), f32)

    out = jnp.zeros((N_PAD, 1), f32)
    prop_cur = prop_pad
    p_cur = prop_pad
    for d in range(DEPTH):
        blk = params["blocks"][d]
        pg = _gather_call(p_cur, gidx).reshape(E_PAD, 2 * D)
        inter = _edge_call(pg, dist2d, *_edge_weights(blk))
        parts = _scatter_call(inter, sidx, zeros_np)
        ow = _out_weights(params["outs"][d])
        if d < DEPTH - 1:
            (pw1, pb1), (pw2, pb2) = params["blocks"][d + 1]["pp"]
            prop_cur, p_cur, out = _node_call(
                parts, prop_cur, out, *ow,
                pw1, pb1.reshape(1, D), pw2, pb2.reshape(1, D),
            )
        else:
            out = _node_last_call(parts, out, *ow)
    return out[:N, 0]
